# baseline (device time: 44018 ns/iter reference)
import jax
import jax.numpy as jnp
from jax import lax
from jax.experimental import pallas as pl
from jax.experimental.pallas import tpu as pltpu

BLKV = 1024
SCALE = 64.0


def kernel(x, W, labels):
    T, D = x.shape
    _, V_shard = W.shape
    Vq = V_shard // 2
    NB = Vq // BLKV
    TS = T // 128

    def w_index(i):
        return (0, lax.axis_index("x") * NB + i)

    def body(x_ref, w_ref, lab_ref, out_ref,
             x8_ref, s_ref, ll_ref,
             csend, crecv_x, crecv_y, crecv_d,
             send_sems, recv_x_sem, recv_y_sem, recv_d_sem):
        i = pl.program_id(0)
        my_x = lax.axis_index("x")
        my_y = lax.axis_index("y")

        @pl.when(i == 0)
        def _():
            x8_ref[...] = x_ref[...].astype(jnp.float8_e4m3fn)
            s_ref[...] = jnp.zeros((T, 1), jnp.float32)
            ll_ref[...] = jnp.zeros((T, 1), jnp.float32)

        w8 = (w_ref[...] * SCALE).astype(jnp.float8_e4m3fn)
        acc = jnp.dot(x8_ref[...], w8, preferred_element_type=jnp.float32)
        s_ref[...] += jnp.sum(jnp.exp(acc * (1.0 / SCALE)),
                              axis=1, keepdims=True)
        local = lab_ref[...] - (my_y * V_shard + my_x * Vq + i * BLKV)
        cols = lax.broadcasted_iota(jnp.int32, (T, BLKV), 1)
        ll_ref[...] += jnp.sum(jnp.where(cols == local, acc, 0.0),
                               axis=1, keepdims=True)

        @pl.when(i == NB - 1)
        def _():
            x_nbr = (1 - my_x, my_y)
            y_nbr = (my_x, 1 - my_y)
            diag = (1 - my_x, 1 - my_y)

            csend[0:TS, :] = s_ref[...].reshape(TS, 128)
            csend[TS:2 * TS, :] = (ll_ref[...] * (1.0 / SCALE)
                                   ).reshape(TS, 128)

            rdmas = []
            for k, (nbr, crecv) in enumerate(
                    ((x_nbr, crecv_x), (y_nbr, crecv_y), (diag, crecv_d))):
                recv_sem = (recv_x_sem, recv_y_sem, recv_d_sem)[k]
                rdma = pltpu.make_async_remote_copy(
                    src_ref=csend, dst_ref=crecv,
                    send_sem=send_sems.at[k], recv_sem=recv_sem,
                    device_id=nbr,
                    device_id_type=pl.DeviceIdType.MESH,
                )
                rdma.start()
                rdmas.append(rdma)
            for rdma in rdmas:
                rdma.wait()

            tot = (csend[...] + crecv_x[...]) + (crecv_y[...] + crecv_d[...])
            out_ref[...] = jnp.log(tot[0:TS, :]) - tot[TS:2 * TS, :]

    out = pl.pallas_call(
        body,
        grid=(NB,),
        out_shape=jax.ShapeDtypeStruct((TS, 128), jnp.float32),
        in_specs=[
            pl.BlockSpec((T, D), lambda i: (0, 0)),
            pl.BlockSpec((D, BLKV), w_index),
            pl.BlockSpec((T, 1), lambda i: (0, 0)),
        ],
        out_specs=pl.BlockSpec((TS, 128), lambda i: (0, 0)),
        scratch_shapes=[
            pltpu.VMEM((T, D), jnp.float8_e4m3fn),
            pltpu.VMEM((T, 1), jnp.float32),
            pltpu.VMEM((T, 1), jnp.float32),
            pltpu.VMEM((2 * TS, 128), jnp.float32),
            pltpu.VMEM((2 * TS, 128), jnp.float32),
            pltpu.VMEM((2 * TS, 128), jnp.float32),
            pltpu.VMEM((2 * TS, 128), jnp.float32),
            pltpu.SemaphoreType.DMA((3,)),
            pltpu.SemaphoreType.DMA,
            pltpu.SemaphoreType.DMA,
            pltpu.SemaphoreType.DMA,
        ],
        compiler_params=pltpu.CompilerParams(
            dimension_semantics=("arbitrary",),
        ),
    )(x, W, labels.reshape(T, 1))
    return out.reshape(T)


# device time: 41442 ns/iter; 1.0622x vs baseline; 1.0622x over previous
import jax
import jax.numpy as jnp
from jax import lax
from jax.experimental import pallas as pl
from jax.experimental.pallas import tpu as pltpu

BLKV = 1024
SCALE = 64.0


def kernel(x, W, labels):
    T, D = x.shape
    _, V_shard = W.shape
    Vq = V_shard // 2
    NB = Vq // BLKV
    TS = T // 128

    def w_index(i):
        return (0, lax.axis_index("x") * NB + i)

    def body(x_ref, w_ref, lab_ref, out_ref,
             x8_ref, s_ref, ll_ref, s2_ref, ll2_ref,
             csend_p, csend_d, crecv_p, crecv_d,
             send_sems, recv_p_sems, recv_d_sems):
        i = pl.program_id(0)
        my_x = lax.axis_index("x")
        my_y = lax.axis_index("y")
        x_nbr = (1 - my_x, my_y)
        y_nbr = (my_x, 1 - my_y)
        diag = (1 - my_x, 1 - my_y)
        peers = (x_nbr, y_nbr, diag)

        @pl.when(i == 0)
        def _():
            x8_ref[...] = x_ref[...].astype(jnp.float8_e4m3fn)
            s_ref[...] = jnp.zeros((T, 1), jnp.float32)
            ll_ref[...] = jnp.zeros((T, 1), jnp.float32)

        @pl.when(i == NB - 1)
        def _():
            barrier = pltpu.get_barrier_semaphore()
            for nbr in peers:
                pl.semaphore_signal(barrier, inc=1, device_id=nbr,
                                    device_id_type=pl.DeviceIdType.MESH)
            pl.semaphore_wait(barrier, 3)

            csend_p[0:TS, :] = s_ref[...].reshape(TS, 128)
            csend_p[TS:2 * TS, :] = (ll_ref[...] * (1.0 / SCALE)
                                     ).reshape(TS, 128)
            for k, nbr in enumerate(peers):
                pltpu.make_async_remote_copy(
                    src_ref=csend_p, dst_ref=crecv_p.at[k],
                    send_sem=send_sems.at[k], recv_sem=recv_p_sems.at[k],
                    device_id=nbr,
                    device_id_type=pl.DeviceIdType.MESH,
                ).start()

        w8 = (w_ref[...] * SCALE).astype(jnp.float8_e4m3fn)
        acc = jnp.dot(x8_ref[...], w8, preferred_element_type=jnp.float32)
        sc = jnp.sum(jnp.exp(acc * (1.0 / SCALE)), axis=1, keepdims=True)
        local = lab_ref[...] - (my_y * V_shard + my_x * Vq + i * BLKV)
        cols = lax.broadcasted_iota(jnp.int32, (T, BLKV), 1)
        llc = jnp.sum(jnp.where(cols == local, acc, 0.0),
                      axis=1, keepdims=True)

        @pl.when(i < NB - 1)
        def _():
            s_ref[...] += sc
            ll_ref[...] += llc

        @pl.when(i == NB - 1)
        def _():
            csend_d[0:TS, :] = sc.reshape(TS, 128)
            csend_d[TS:2 * TS, :] = (llc * (1.0 / SCALE)).reshape(TS, 128)
            rdmas = []
            for k, nbr in enumerate(peers):
                rdma = pltpu.make_async_remote_copy(
                    src_ref=csend_d, dst_ref=crecv_d.at[k],
                    send_sem=send_sems.at[3 + k], recv_sem=recv_d_sems.at[k],
                    device_id=nbr,
                    device_id_type=pl.DeviceIdType.MESH,
                )
                rdma.start()
                rdmas.append(rdma)

            for k, nbr in enumerate(peers):
                pltpu.make_async_remote_copy(
                    src_ref=csend_p, dst_ref=crecv_p.at[k],
                    send_sem=send_sems.at[k], recv_sem=recv_p_sems.at[k],
                    device_id=nbr,
                    device_id_type=pl.DeviceIdType.MESH,
                ).wait()
            for rdma in rdmas:
                rdma.wait()

            tot = ((csend_p[...] + csend_d[...])
                   + (crecv_p[0] + crecv_d[0])
                   + (crecv_p[1] + crecv_d[1])
                   + (crecv_p[2] + crecv_d[2]))
            out_ref[...] = jnp.log(tot[0:TS, :]) - tot[TS:2 * TS, :]

    out = pl.pallas_call(
        body,
        grid=(NB,),
        out_shape=jax.ShapeDtypeStruct((TS, 128), jnp.float32),
        in_specs=[
            pl.BlockSpec((T, D), lambda i: (0, 0)),
            pl.BlockSpec((D, BLKV), w_index),
            pl.BlockSpec((T, 1), lambda i: (0, 0)),
        ],
        out_specs=pl.BlockSpec((TS, 128), lambda i: (0, 0)),
        scratch_shapes=[
            pltpu.VMEM((T, D), jnp.float8_e4m3fn),
            pltpu.VMEM((T, 1), jnp.float32),
            pltpu.VMEM((T, 1), jnp.float32),
            pltpu.VMEM((T, 1), jnp.float32),
            pltpu.VMEM((T, 1), jnp.float32),
            pltpu.VMEM((2 * TS, 128), jnp.float32),
            pltpu.VMEM((2 * TS, 128), jnp.float32),
            pltpu.VMEM((3, 2 * TS, 128), jnp.float32),
            pltpu.VMEM((3, 2 * TS, 128), jnp.float32),
            pltpu.SemaphoreType.DMA((6,)),
            pltpu.SemaphoreType.DMA((3,)),
            pltpu.SemaphoreType.DMA((3,)),
        ],
        compiler_params=pltpu.CompilerParams(
            collective_id=0,
            dimension_semantics=("arbitrary",),
        ),
    )(x, W, labels.reshape(T, 1))
    return out.reshape(T)


# device time: 41030 ns/iter; 1.0728x vs baseline; 1.0100x over previous
import jax
import jax.numpy as jnp
from jax import lax
from jax.experimental import pallas as pl
from jax.experimental.pallas import tpu as pltpu

BLKV = 1024
SCALE = 64.0


def kernel(x, W, labels):
    T, D = x.shape
    _, V_shard = W.shape
    Vq = V_shard // 2
    NB = Vq // BLKV
    TS = T // 128

    def w_index(i):
        return (0, lax.axis_index("x") * NB + i)

    def body(x_ref, w_ref, lab_ref, out_ref,
             x8_ref, s_ref, ll_ref,
             csend, crecv_x, crecv_y, crecv_d,
             send_sems, recv_x_sem, recv_y_sem, recv_d_sem):
        i = pl.program_id(0)
        my_x = lax.axis_index("x")
        my_y = lax.axis_index("y")

        @pl.when(i == 0)
        def _():
            x8_ref[...] = x_ref[...].astype(jnp.float8_e4m3fn)
            s_ref[...] = jnp.zeros((T, 1), jnp.float32)
            ll_ref[...] = jnp.zeros((T, 1), jnp.float32)

        w8 = (w_ref[...] * SCALE).astype(jnp.float8_e4m3fn)
        acc = jnp.dot(x8_ref[...], w8, preferred_element_type=jnp.float32)
        s_ref[...] += jnp.sum(jnp.exp(acc * (1.0 / SCALE)),
                              axis=1, keepdims=True)
        local = lab_ref[...] - (my_y * V_shard + my_x * Vq + i * BLKV)
        cols = lax.broadcasted_iota(jnp.int32, (T, BLKV), 1)
        ll_ref[...] += jnp.sum(jnp.where(cols == local, acc, 0.0),
                               axis=1, keepdims=True)

        @pl.when(i == NB - 1)
        def _():
            x_nbr = (1 - my_x, my_y)
            y_nbr = (my_x, 1 - my_y)
            diag = (1 - my_x, 1 - my_y)
            barrier = pltpu.get_barrier_semaphore()
            for nbr in (x_nbr, y_nbr, diag):
                pl.semaphore_signal(barrier, inc=1, device_id=nbr,
                                    device_id_type=pl.DeviceIdType.MESH)
            pl.semaphore_wait(barrier, 3)

            csend[0:TS, :] = s_ref[...].reshape(TS, 128)
            csend[TS:2 * TS, :] = (ll_ref[...] * (1.0 / SCALE)
                                   ).reshape(TS, 128)

            rdmas = []
            for k, (nbr, crecv) in enumerate(
                    ((x_nbr, crecv_x), (y_nbr, crecv_y), (diag, crecv_d))):
                recv_sem = (recv_x_sem, recv_y_sem, recv_d_sem)[k]
                rdma = pltpu.make_async_remote_copy(
                    src_ref=csend, dst_ref=crecv,
                    send_sem=send_sems.at[k], recv_sem=recv_sem,
                    device_id=nbr,
                    device_id_type=pl.DeviceIdType.MESH,
                )
                rdma.start()
                rdmas.append(rdma)
            for rdma in rdmas:
                rdma.wait()

            tot = (csend[...] + crecv_x[...]) + (crecv_y[...] + crecv_d[...])
            out_ref[...] = jnp.log(tot[0:TS, :]) - tot[TS:2 * TS, :]

    out = pl.pallas_call(
        body,
        grid=(NB,),
        out_shape=jax.ShapeDtypeStruct((TS, 128), jnp.float32),
        in_specs=[
            pl.BlockSpec((T, D), lambda i: (0, 0)),
            pl.BlockSpec((D, BLKV), w_index),
            pl.BlockSpec((T, 1), lambda i: (0, 0)),
        ],
        out_specs=pl.BlockSpec((TS, 128), lambda i: (0, 0)),
        scratch_shapes=[
            pltpu.VMEM((T, D), jnp.float8_e4m3fn),
            pltpu.VMEM((T, 1), jnp.float32),
            pltpu.VMEM((T, 1), jnp.float32),
            pltpu.VMEM((2 * TS, 128), jnp.float32),
            pltpu.VMEM((2 * TS, 128), jnp.float32),
            pltpu.VMEM((2 * TS, 128), jnp.float32),
            pltpu.VMEM((2 * TS, 128), jnp.float32),
            pltpu.SemaphoreType.DMA((3,)),
            pltpu.SemaphoreType.DMA,
            pltpu.SemaphoreType.DMA,
            pltpu.SemaphoreType.DMA,
        ],
        compiler_params=pltpu.CompilerParams(
            collective_id=0,
            dimension_semantics=("arbitrary",),
        ),
    )(x, W, labels.reshape(T, 1))
    return out.reshape(T)


# device time: 40849 ns/iter; 1.0776x vs baseline; 1.0044x over previous
import jax
import jax.numpy as jnp
from jax import lax
from jax.experimental import pallas as pl
from jax.experimental.pallas import tpu as pltpu

BLKV = 1024
SCALE = 64.0


def kernel(x, W, labels):
    T, D = x.shape
    _, V_shard = W.shape
    Vq = V_shard // 2
    NB = Vq // BLKV
    TS = T // 128

    def w_index(i):
        return (0, lax.axis_index("x") * NB + i)

    def body(x_ref, w_ref, lab_ref, out_ref,
             x8_ref, s_ref, ll_ref,
             csend, crecv_x, crecv_y, crecv_d,
             send_sems, recv_x_sem, recv_y_sem, recv_d_sem):
        i = pl.program_id(0)
        my_x = lax.axis_index("x")
        my_y = lax.axis_index("y")

        @pl.when(i == 0)
        def _():
            x8_ref[...] = x_ref[...].astype(jnp.float8_e4m3fn)
            s_ref[...] = jnp.zeros((T, 1), jnp.float32)
            ll_ref[...] = jnp.zeros((T, 1), jnp.float32)

        w8 = (w_ref[...] * SCALE).astype(jnp.float8_e4m3fn)
        acc = jnp.dot(x8_ref[...], w8, preferred_element_type=jnp.float32)
        s_ref[...] += jnp.sum(jnp.exp(acc * (1.0 / SCALE)),
                              axis=1, keepdims=True)
        local = lab_ref[...] - (my_y * V_shard + my_x * Vq + i * BLKV)
        cols = lax.broadcasted_iota(jnp.int32, (T, BLKV), 1)
        ll_ref[...] += jnp.sum(jnp.where(cols == local, acc, 0.0),
                               axis=1, keepdims=True)

        @pl.when(i == NB - 2)
        def _():
            barrier = pltpu.get_barrier_semaphore()
            for nbr in ((1 - my_x, my_y), (my_x, 1 - my_y),
                        (1 - my_x, 1 - my_y)):
                pl.semaphore_signal(barrier, inc=1, device_id=nbr,
                                    device_id_type=pl.DeviceIdType.MESH)
            pl.semaphore_wait(barrier, 3)

        @pl.when(i == NB - 1)
        def _():
            x_nbr = (1 - my_x, my_y)
            y_nbr = (my_x, 1 - my_y)
            diag = (1 - my_x, 1 - my_y)

            csend[0:TS, :] = s_ref[...].reshape(TS, 128)
            csend[TS:2 * TS, :] = (ll_ref[...] * (1.0 / SCALE)
                                   ).reshape(TS, 128)

            rdmas = []
            for k, (nbr, crecv) in enumerate(
                    ((x_nbr, crecv_x), (y_nbr, crecv_y), (diag, crecv_d))):
                recv_sem = (recv_x_sem, recv_y_sem, recv_d_sem)[k]
                rdma = pltpu.make_async_remote_copy(
                    src_ref=csend, dst_ref=crecv,
                    send_sem=send_sems.at[k], recv_sem=recv_sem,
                    device_id=nbr,
                    device_id_type=pl.DeviceIdType.MESH,
                )
                rdma.start()
                rdmas.append(rdma)
            for rdma in rdmas:
                rdma.wait()

            tot = (csend[...] + crecv_x[...]) + (crecv_y[...] + crecv_d[...])
            out_ref[...] = jnp.log(tot[0:TS, :]) - tot[TS:2 * TS, :]

    out = pl.pallas_call(
        body,
        grid=(NB,),
        out_shape=jax.ShapeDtypeStruct((TS, 128), jnp.float32),
        in_specs=[
            pl.BlockSpec((T, D), lambda i: (0, 0)),
            pl.BlockSpec((D, BLKV), w_index),
            pl.BlockSpec((T, 1), lambda i: (0, 0)),
        ],
        out_specs=pl.BlockSpec((TS, 128), lambda i: (0, 0)),
        scratch_shapes=[
            pltpu.VMEM((T, D), jnp.float8_e4m3fn),
            pltpu.VMEM((T, 1), jnp.float32),
            pltpu.VMEM((T, 1), jnp.float32),
            pltpu.VMEM((2 * TS, 128), jnp.float32),
            pltpu.VMEM((2 * TS, 128), jnp.float32),
            pltpu.VMEM((2 * TS, 128), jnp.float32),
            pltpu.VMEM((2 * TS, 128), jnp.float32),
            pltpu.SemaphoreType.DMA((3,)),
            pltpu.SemaphoreType.DMA,
            pltpu.SemaphoreType.DMA,
            pltpu.SemaphoreType.DMA,
        ],
        compiler_params=pltpu.CompilerParams(
            collective_id=0,
            dimension_semantics=("arbitrary",),
        ),
    )(x, W, labels.reshape(T, 1))
    return out.reshape(T)
